# Initial kernel scaffold; baseline (speedup 1.0000x reference)
import jax
import jax.numpy as jnp
from jax.experimental import pallas as pl


def kernel(points, point_features, boxes3d):
    raise NotImplementedError("write your pallas kernel here")


# speedup vs baseline: 9.2827x; 9.2827x over previous
import functools

import jax
import jax.numpy as jnp
from jax import lax
from jax.experimental import pallas as pl
from jax.experimental.pallas import tpu as pltpu
from jax.experimental.pallas import tpu_sc as plsc

_NUM_SAMPLED = 512
_LANES = 16


def _make_sc_kernel(B, N, C, M, S):
    D = 3 + C
    NW = 32
    BOXES_PER_W = (B * M) // NW
    W_PER_B = NW // B
    N1 = N + 1
    PIECE = 128
    ROWLEN = PIECE * D
    mesh = plsc.VectorSubcoreMesh(core_axis_name="c", subcore_axis_name="s")

    @functools.partial(
        pl.kernel,
        out_type=[
            jax.ShapeDtypeStruct((B * M, S * D), jnp.float32),
            jax.ShapeDtypeStruct((B * M,), jnp.int32),
        ],
        mesh=mesh,
        compiler_params=pltpu.CompilerParams(needs_layout_passes=False),
        scratch_types=[
            pltpu.VMEM((3, N), jnp.float32),
            pltpu.VMEM((BOXES_PER_W, _LANES), jnp.float32),
            pltpu.VMEM((S,), jnp.int32),
            pltpu.VMEM((1, PIECE), jnp.int32),
            pltpu.VMEM((PIECE, C), jnp.float32),
            pltpu.VMEM((ROWLEN,), jnp.float32),
            pltpu.VMEM((BOXES_PER_W,), jnp.int32),
            pltpu.SemaphoreType.DMA,
        ],
    )
    def sc_kernel(feat_hbm, ptsT_hbm, boxp_hbm, out_hbm, flag_hbm,
                  xyz_v, boxp_v, idx_v, exp_v, feat_v, rows_v, flags_v, sem):
        wid = lax.axis_index("s") * 2 + lax.axis_index("c")
        b = wid // W_PER_B
        box0 = wid * BOXES_PER_W

        pltpu.sync_copy(ptsT_hbm.at[b], xyz_v)
        pltpu.sync_copy(boxp_hbm.at[pl.ds(box0, BOXES_PER_W)], boxp_v)

        lane = lax.iota(jnp.int32, _LANES)
        zero_row = b * N1 + N

        def box_body(j, flags):
            prow = boxp_v[j, :]
            cx = prow[0]
            cy = prow[1]
            cz = prow[2]
            dx2 = prow[3]
            dy2 = prow[4]
            dz2 = prow[5]
            ca = prow[6]
            sa = prow[7]

            def scan_body(ii, off):
                base = ii * _LANES
                x = xyz_v[0, pl.ds(base, _LANES)]
                y = xyz_v[1, pl.ds(base, _LANES)]
                z = xyz_v[2, pl.ds(base, _LANES)]
                sx = x - cx
                sy = y - cy
                lx = sx * ca + sy * (-sa)
                ly = sx * sa + sy * ca
                m = (
                    (jnp.abs(z - cz) <= dz2)
                    & (lx > -dx2) & (lx < dx2)
                    & (ly > -dy2) & (ly < dy2)
                )
                cs = plsc.cumsum(m.astype(jnp.int32))
                pos = off + cs - 1
                plsc.store_scatter(idx_v, [pos], base + lane,
                                   mask=m & (pos < S))
                return off + plsc.all_reduce_population_count(m)

            cnt = lax.fori_loop(0, N // _LANES, scan_body,
                                jnp.zeros((_LANES,), jnp.int32))
            cnt_safe = jnp.maximum(cnt, 1)
            nonempty = cnt > 0

            for q in range(S // PIECE):
                for r in range(0, PIECE, _LANES):
                    ar = q * PIECE + r + lane
                    p = jnp.where(ar < cnt, ar, lax.rem(ar, cnt_safe))
                    g = plsc.load_gather(idx_v, [p])
                    g = jnp.where(nonempty, g, 0)
                    exp_v[0, pl.ds(r, _LANES)] = jnp.where(
                        nonempty, b * N1 + g, zero_row)
                    slot0 = (r + lane) * D
                    for d in range(3):
                        xv = plsc.load_gather(
                            xyz_v, [jnp.full((_LANES,), d, jnp.int32), g])
                        xv = jnp.where(nonempty, xv, 0.0)
                        plsc.store_scatter(rows_v, [slot0 + d], xv)

                pltpu.async_copy(feat_hbm.at[exp_v.at[0]], feat_v, sem).wait()

                def ileave(r, _):
                    base = r * D + 3
                    for k in range(0, C, _LANES):
                        rows_v[pl.ds(base + k, _LANES)] = \
                            feat_v[r, pl.ds(k, _LANES)]
                    return 0

                lax.fori_loop(0, PIECE, ileave, 0)
                pltpu.sync_copy(
                    rows_v, out_hbm.at[box0 + j, pl.ds(q * ROWLEN, ROWLEN)])

            return jnp.where(lane == j, (~nonempty).astype(jnp.int32), flags)

        flags = lax.fori_loop(0, BOXES_PER_W, box_body,
                              jnp.zeros((_LANES,), jnp.int32))
        flags_v[...] = flags
        pltpu.sync_copy(flags_v, flag_hbm.at[pl.ds(box0, BOXES_PER_W)])

    return sc_kernel


def kernel(points, point_features, boxes3d):
    B, N, _ = points.shape
    C = point_features.shape[2]
    M = boxes3d.shape[1]
    S = _NUM_SAMPLED

    feat = jnp.concatenate(
        [point_features, jnp.zeros((B, 1, C), jnp.float32)], axis=1)
    feat = feat.reshape(B * (N + 1), C)
    ptsT = jnp.transpose(points, (0, 2, 1))

    cx = boxes3d[..., 0]
    cy = boxes3d[..., 1]
    dz = boxes3d[..., 5]
    cz = boxes3d[..., 2] + dz / 2.0
    dx2 = boxes3d[..., 3] / 2.0
    dy2 = boxes3d[..., 4] / 2.0
    dz2 = dz / 2.0
    ca = jnp.cos(-boxes3d[..., 6])
    sa = jnp.sin(-boxes3d[..., 6])
    zz = jnp.zeros_like(cx)
    boxp = jnp.stack([cx, cy, cz, dx2, dy2, dz2, ca, sa,
                      zz, zz, zz, zz, zz, zz, zz, zz],
                     axis=-1).reshape(B * M, _LANES)

    sc = _make_sc_kernel(B, N, C, M, S)
    pooled, flag = sc(feat, ptsT, boxp)
    return pooled.reshape(B, M, S, 3 + C), flag.reshape(B, M)


# speedup vs baseline: 11.1713x; 1.2035x over previous
import functools

import jax
import jax.numpy as jnp
from jax import lax
from jax.experimental import pallas as pl
from jax.experimental.pallas import tpu as pltpu
from jax.experimental.pallas import tpu_sc as plsc

_NUM_SAMPLED = 512
_LANES = 16


def _make_sc_kernel(B, N, C, M, S):
    D = 3 + C
    NW = 32
    BOXES_PER_W = (B * M) // NW
    W_PER_B = NW // B
    PIECE = 128
    ROWLEN = PIECE * D
    mesh = plsc.VectorSubcoreMesh(core_axis_name="c", subcore_axis_name="s")

    @functools.partial(
        pl.kernel,
        out_type=[
            jax.ShapeDtypeStruct((B * M, S * D), jnp.float32),
            jax.ShapeDtypeStruct((B * M,), jnp.int32),
        ],
        mesh=mesh,
        compiler_params=pltpu.CompilerParams(needs_layout_passes=False),
        scratch_types=[
            pltpu.VMEM((3, N), jnp.float32),
            pltpu.VMEM((BOXES_PER_W, _LANES), jnp.float32),
            pltpu.VMEM((S,), jnp.int32),
            pltpu.VMEM((1, PIECE), jnp.int32),
            pltpu.VMEM((PIECE, C), jnp.float32),
            pltpu.VMEM((ROWLEN,), jnp.float32),
            pltpu.VMEM((BOXES_PER_W,), jnp.int32),
            pltpu.SemaphoreType.DMA,
        ],
    )
    def sc_kernel(feat_hbm, ptsT_hbm, boxp_hbm, out_hbm, flag_hbm,
                  xyz_v, boxp_v, idx_v, exp_v, feat_v, rows_v, flags_v, sem):
        wid = lax.axis_index("s") * 2 + lax.axis_index("c")
        b = wid // W_PER_B
        box0 = wid * BOXES_PER_W

        pltpu.sync_copy(ptsT_hbm.at[b], xyz_v)
        pltpu.sync_copy(boxp_hbm.at[pl.ds(box0, BOXES_PER_W)], boxp_v)

        lane = lax.iota(jnp.int32, _LANES)

        def box_body(j, flags):
            prow = boxp_v[j, :]
            cx = prow[0]
            cy = prow[1]
            cz = prow[2]
            dx2 = prow[3]
            dy2 = prow[4]
            dz2 = prow[5]
            ca = prow[6]
            sa = prow[7]

            def test(base):
                x = xyz_v[0, pl.ds(base, _LANES)]
                y = xyz_v[1, pl.ds(base, _LANES)]
                z = xyz_v[2, pl.ds(base, _LANES)]
                sx = x - cx
                sy = y - cy
                lx = sx * ca + sy * (-sa)
                ly = sx * sa + sy * ca
                return (
                    (jnp.abs(z - cz) <= dz2)
                    & (lx > -dx2) & (lx < dx2)
                    & (ly > -dy2) & (ly < dy2)
                )

            def scan_body(ii, off):
                base = ii * (2 * _LANES)
                m0 = test(base)
                m1 = test(base + _LANES)
                cs0 = plsc.cumsum(m0.astype(jnp.int32))
                cs1 = plsc.cumsum(m1.astype(jnp.int32))
                pc0 = plsc.all_reduce_population_count(m0)
                pc1 = plsc.all_reduce_population_count(m1)
                pos0 = off + cs0 - 1
                pos1 = off + pc0 + cs1 - 1
                plsc.store_scatter(idx_v, [pos0], base + lane,
                                   mask=m0 & (pos0 < S))
                plsc.store_scatter(idx_v, [pos1], base + _LANES + lane,
                                   mask=m1 & (pos1 < S))
                return off + (pc0 + pc1)

            cnt = lax.fori_loop(0, N // (2 * _LANES), scan_body,
                                jnp.zeros((_LANES,), jnp.int32))
            cnt_safe = jnp.maximum(cnt, 1)
            nonempty = cnt[0] > 0

            @pl.when(nonempty)
            def _pooled():
                for q in range(S // PIECE):
                    for r in range(0, PIECE, _LANES):
                        ar = q * PIECE + r + lane
                        p = jnp.where(ar < cnt, ar, lax.rem(ar, cnt_safe))
                        g = plsc.load_gather(idx_v, [p])
                        exp_v[0, pl.ds(r, _LANES)] = b * N + g
                        slot0 = (r + lane) * D
                        for d in range(3):
                            xv = plsc.load_gather(
                                xyz_v, [jnp.full((_LANES,), d, jnp.int32), g])
                            plsc.store_scatter(rows_v, [slot0 + d], xv)

                    pltpu.async_copy(feat_hbm.at[exp_v.at[0]],
                                     feat_v, sem).wait()

                    def ileave(r, _):
                        base = r * D + 3
                        for k in range(0, C, _LANES):
                            rows_v[pl.ds(base + k, _LANES)] = \
                                feat_v[r, pl.ds(k, _LANES)]
                        return 0

                    lax.fori_loop(0, PIECE, ileave, 0)
                    pltpu.sync_copy(
                        rows_v,
                        out_hbm.at[box0 + j, pl.ds(q * ROWLEN, ROWLEN)])

            @pl.when(jnp.logical_not(nonempty))
            def _zeros():
                zv = jnp.zeros((_LANES,), jnp.float32)

                def zbody(r, _):
                    rows_v[pl.ds(r * _LANES, _LANES)] = zv
                    return 0

                lax.fori_loop(0, ROWLEN // _LANES, zbody, 0)
                for q in range(S // PIECE):
                    pltpu.sync_copy(
                        rows_v,
                        out_hbm.at[box0 + j, pl.ds(q * ROWLEN, ROWLEN)])

            return jnp.where(lane == j, 1 - (cnt > 0).astype(jnp.int32),
                             flags)

        flags = lax.fori_loop(0, BOXES_PER_W, box_body,
                              jnp.zeros((_LANES,), jnp.int32))
        flags_v[...] = flags
        pltpu.sync_copy(flags_v, flag_hbm.at[pl.ds(box0, BOXES_PER_W)])

    return sc_kernel


def kernel(points, point_features, boxes3d):
    B, N, _ = points.shape
    C = point_features.shape[2]
    M = boxes3d.shape[1]
    S = _NUM_SAMPLED

    feat = point_features.reshape(B * N, C)
    ptsT = jnp.transpose(points, (0, 2, 1))

    cx = boxes3d[..., 0]
    cy = boxes3d[..., 1]
    dz = boxes3d[..., 5]
    cz = boxes3d[..., 2] + dz / 2.0
    dx2 = boxes3d[..., 3] / 2.0
    dy2 = boxes3d[..., 4] / 2.0
    dz2 = dz / 2.0
    ca = jnp.cos(-boxes3d[..., 6])
    sa = jnp.sin(-boxes3d[..., 6])
    zz = jnp.zeros_like(cx)
    boxp = jnp.stack([cx, cy, cz, dx2, dy2, dz2, ca, sa,
                      zz, zz, zz, zz, zz, zz, zz, zz],
                     axis=-1).reshape(B * M, _LANES)

    sc = _make_sc_kernel(B, N, C, M, S)
    pooled, flag = sc(feat, ptsT, boxp)
    return pooled.reshape(B, M, S, 3 + C), flag.reshape(B, M)


# speedup vs baseline: 12.1965x; 1.0918x over previous
import functools

import jax
import jax.numpy as jnp
from jax import lax
from jax.experimental import pallas as pl
from jax.experimental.pallas import tpu as pltpu
from jax.experimental.pallas import tpu_sc as plsc

_NUM_SAMPLED = 512
_LANES = 16


def _make_sc_kernel(B, N, C, M, S):
    D = 3 + C
    NW = 32
    BOXES_PER_W = (B * M) // NW
    W_PER_B = NW // B
    PIECE = 128
    ROWLEN = PIECE * D
    mesh = plsc.VectorSubcoreMesh(core_axis_name="c", subcore_axis_name="s")

    @functools.partial(
        pl.kernel,
        out_type=[
            jax.ShapeDtypeStruct((B * M, S * D), jnp.float32),
            jax.ShapeDtypeStruct((B * M,), jnp.int32),
        ],
        mesh=mesh,
        compiler_params=pltpu.CompilerParams(needs_layout_passes=False),
        scratch_types=[
            pltpu.VMEM((3, N), jnp.float32),
            pltpu.VMEM((BOXES_PER_W, _LANES), jnp.float32),
            pltpu.VMEM((1, _LANES), jnp.float32),
            pltpu.VMEM((N + 2 * _LANES,), jnp.int32),
            pltpu.VMEM((S,), jnp.int32),
            pltpu.VMEM((1, PIECE), jnp.int32),
            pltpu.VMEM((PIECE, C), jnp.float32),
            pltpu.VMEM((ROWLEN,), jnp.float32),
            pltpu.VMEM((BOXES_PER_W,), jnp.int32),
            pltpu.SemaphoreType.DMA,
        ],
    )
    def sc_kernel(feat_hbm, ptsT_hbm, boxp_hbm, wb_hbm, out_hbm, flag_hbm,
                  xyz_v, boxp_v, wb_v, filt_v, idx_v, exp_v, feat_v, rows_v,
                  flags_v, sem):
        wid = lax.axis_index("s") * 2 + lax.axis_index("c")
        b = wid // W_PER_B
        box0 = wid * BOXES_PER_W

        pltpu.sync_copy(ptsT_hbm.at[b], xyz_v)
        pltpu.sync_copy(boxp_hbm.at[pl.ds(box0, BOXES_PER_W)], boxp_v)
        pltpu.sync_copy(wb_hbm.at[pl.ds(wid, 1)], wb_v)

        lane = lax.iota(jnp.int32, _LANES)

        wrow = wb_v[0, :]
        xlo, xhi = wrow[0], wrow[1]
        ylo, yhi = wrow[2], wrow[3]
        zlo, zhi = wrow[4], wrow[5]

        def bb_test(base):
            x = xyz_v[0, pl.ds(base, _LANES)]
            y = xyz_v[1, pl.ds(base, _LANES)]
            z = xyz_v[2, pl.ds(base, _LANES)]
            return ((x >= xlo) & (x <= xhi) & (y >= ylo) & (y <= yhi)
                    & (z >= zlo) & (z <= zhi))

        def filt_body(ii, off):
            base = ii * (2 * _LANES)
            m0 = bb_test(base)
            m1 = bb_test(base + _LANES)
            cs0 = plsc.cumsum(m0.astype(jnp.int32))
            cs1 = plsc.cumsum(m1.astype(jnp.int32))
            pc0 = plsc.all_reduce_population_count(m0)
            pc1 = plsc.all_reduce_population_count(m1)
            plsc.store_scatter(filt_v, [off + cs0 - 1], base + lane, mask=m0)
            plsc.store_scatter(filt_v, [off + pc0 + cs1 - 1],
                               base + _LANES + lane, mask=m1)
            return off + (pc0 + pc1)

        nf = lax.fori_loop(0, N // (2 * _LANES), filt_body,
                           jnp.zeros((_LANES,), jnp.int32))
        plsc.store_scatter(filt_v, [nf + lane], jnp.zeros((_LANES,), jnp.int32))
        plsc.store_scatter(filt_v, [nf + _LANES + lane],
                           jnp.zeros((_LANES,), jnp.int32))
        nf_s = nf[0]
        ntrip = (nf_s + 2 * _LANES - 1) // (2 * _LANES)

        def box_body(j, flags):
            prow = boxp_v[j, :]
            cx = prow[0]
            cy = prow[1]
            cz = prow[2]
            dx2 = prow[3]
            dy2 = prow[4]
            dz2 = prow[5]
            ca = prow[6]
            sa = prow[7]

            def test(base):
                fc = filt_v[pl.ds(base, _LANES)]
                x = plsc.load_gather(
                    xyz_v, [jnp.zeros((_LANES,), jnp.int32), fc])
                y = plsc.load_gather(
                    xyz_v, [jnp.ones((_LANES,), jnp.int32), fc])
                z = plsc.load_gather(
                    xyz_v, [jnp.full((_LANES,), 2, jnp.int32), fc])
                sx = x - cx
                sy = y - cy
                lx = sx * ca + sy * (-sa)
                ly = sx * sa + sy * ca
                m = (
                    (jnp.abs(z - cz) <= dz2)
                    & (lx > -dx2) & (lx < dx2)
                    & (ly > -dy2) & (ly < dy2)
                    & (base + lane < nf)
                )
                return fc, m

            def scan_body(ii, off):
                base = ii * (2 * _LANES)
                fc0, m0 = test(base)
                fc1, m1 = test(base + _LANES)
                cs0 = plsc.cumsum(m0.astype(jnp.int32))
                cs1 = plsc.cumsum(m1.astype(jnp.int32))
                pc0 = plsc.all_reduce_population_count(m0)
                pc1 = plsc.all_reduce_population_count(m1)
                pos0 = off + cs0 - 1
                pos1 = off + pc0 + cs1 - 1
                plsc.store_scatter(idx_v, [pos0], fc0,
                                   mask=m0 & (pos0 < S))
                plsc.store_scatter(idx_v, [pos1], fc1,
                                   mask=m1 & (pos1 < S))
                return off + (pc0 + pc1)

            cnt = lax.fori_loop(0, ntrip, scan_body,
                                jnp.zeros((_LANES,), jnp.int32))
            cnt_safe = jnp.maximum(cnt, 1)
            nonempty = cnt[0] > 0

            @pl.when(nonempty)
            def _pooled():
                for q in range(S // PIECE):
                    for r in range(0, PIECE, _LANES):
                        ar = q * PIECE + r + lane
                        p = jnp.where(ar < cnt, ar, lax.rem(ar, cnt_safe))
                        g = plsc.load_gather(idx_v, [p])
                        exp_v[0, pl.ds(r, _LANES)] = b * N + g
                        slot0 = (r + lane) * D
                        for d in range(3):
                            xv = plsc.load_gather(
                                xyz_v, [jnp.full((_LANES,), d, jnp.int32), g])
                            plsc.store_scatter(rows_v, [slot0 + d], xv)

                    pltpu.async_copy(feat_hbm.at[exp_v.at[0]],
                                     feat_v, sem).wait()

                    def ileave(r, _):
                        base = r * D + 3
                        for k in range(0, C, _LANES):
                            rows_v[pl.ds(base + k, _LANES)] = \
                                feat_v[r, pl.ds(k, _LANES)]
                        return 0

                    lax.fori_loop(0, PIECE, ileave, 0)
                    pltpu.sync_copy(
                        rows_v,
                        out_hbm.at[box0 + j, pl.ds(q * ROWLEN, ROWLEN)])

            @pl.when(jnp.logical_not(nonempty))
            def _zeros():
                zv = jnp.zeros((_LANES,), jnp.float32)

                def zbody(r, _):
                    rows_v[pl.ds(r * _LANES, _LANES)] = zv
                    return 0

                lax.fori_loop(0, ROWLEN // _LANES, zbody, 0)
                for q in range(S // PIECE):
                    pltpu.sync_copy(
                        rows_v,
                        out_hbm.at[box0 + j, pl.ds(q * ROWLEN, ROWLEN)])

            return jnp.where(lane == j, 1 - (cnt > 0).astype(jnp.int32),
                             flags)

        flags = lax.fori_loop(0, BOXES_PER_W, box_body,
                              jnp.zeros((_LANES,), jnp.int32))
        flags_v[...] = flags
        pltpu.sync_copy(flags_v, flag_hbm.at[pl.ds(box0, BOXES_PER_W)])

    return sc_kernel


def kernel(points, point_features, boxes3d):
    B, N, _ = points.shape
    C = point_features.shape[2]
    M = boxes3d.shape[1]
    S = _NUM_SAMPLED

    feat = point_features.reshape(B * N, C)
    ptsT = jnp.transpose(points, (0, 2, 1))

    cx = boxes3d[..., 0]
    cy = boxes3d[..., 1]
    dz = boxes3d[..., 5]
    cz = boxes3d[..., 2] + dz / 2.0
    dx2 = boxes3d[..., 3] / 2.0
    dy2 = boxes3d[..., 4] / 2.0
    dz2 = dz / 2.0
    ca = jnp.cos(-boxes3d[..., 6])
    sa = jnp.sin(-boxes3d[..., 6])
    zz = jnp.zeros_like(cx)
    boxp = jnp.stack([cx, cy, cz, dx2, dy2, dz2, ca, sa,
                      zz, zz, zz, zz, zz, zz, zz, zz],
                     axis=-1).reshape(B * M, _LANES)

    NW = 32
    r = (dx2 + dy2).reshape(NW, -1) * 1.001 + 1e-5
    cxw = cx.reshape(NW, -1)
    cyw = cy.reshape(NW, -1)
    czw = cz.reshape(NW, -1)
    dzw = dz2.reshape(NW, -1) * 1.001 + 1e-5
    xlo = jnp.min(cxw - r, axis=1)
    xhi = jnp.max(cxw + r, axis=1)
    ylo = jnp.min(cyw - r, axis=1)
    yhi = jnp.max(cyw + r, axis=1)
    zlo = jnp.min(czw - dzw, axis=1)
    zhi = jnp.max(czw + dzw, axis=1)
    zw = jnp.zeros_like(xlo)
    wb = jnp.stack([xlo, xhi, ylo, yhi, zlo, zhi,
                    zw, zw, zw, zw, zw, zw, zw, zw, zw, zw],
                   axis=-1)

    sc = _make_sc_kernel(B, N, C, M, S)
    pooled, flag = sc(feat, ptsT, boxp, wb)
    return pooled.reshape(B, M, S, 3 + C), flag.reshape(B, M)


# speedup vs baseline: 25.7179x; 2.1086x over previous
import functools

import jax
import jax.numpy as jnp
from jax import lax
from jax.experimental import pallas as pl
from jax.experimental.pallas import tpu as pltpu
from jax.experimental.pallas import tpu_sc as plsc

_NUM_SAMPLED = 512
_LANES = 16


def _make_sc_kernel(B, N, C, M, S):
    D = 3 + C
    NW = 32
    BOXES_PER_W = (B * M) // NW
    W_PER_B = NW // B
    PIECE = 64
    NP = S // PIECE
    mesh = plsc.VectorSubcoreMesh(core_axis_name="c", subcore_axis_name="s")

    @functools.partial(
        pl.kernel,
        out_type=[
            jax.ShapeDtypeStruct((B * M, S, C), jnp.float32),
            jax.ShapeDtypeStruct((B * M, S * 3), jnp.float32),
            jax.ShapeDtypeStruct((B * M,), jnp.int32),
        ],
        mesh=mesh,
        compiler_params=pltpu.CompilerParams(needs_layout_passes=False),
        scratch_types=[
            pltpu.VMEM((3, N), jnp.float32),
            pltpu.VMEM((BOXES_PER_W, _LANES), jnp.float32),
            pltpu.VMEM((1, _LANES), jnp.float32),
            pltpu.VMEM((N + 2 * _LANES,), jnp.int32),
            pltpu.VMEM((S,), jnp.int32),
            pltpu.VMEM((S // PIECE, PIECE), jnp.int32),
            pltpu.VMEM((4, PIECE, C), jnp.float32),
            pltpu.VMEM((S * 3,), jnp.float32),
            pltpu.VMEM((BOXES_PER_W,), jnp.int32),
            pltpu.SemaphoreType.DMA,
            pltpu.SemaphoreType.DMA,
            pltpu.SemaphoreType.DMA,
            pltpu.SemaphoreType.DMA,
            pltpu.SemaphoreType.DMA,
            pltpu.SemaphoreType.DMA,
            pltpu.SemaphoreType.DMA,
            pltpu.SemaphoreType.DMA,
        ],
    )
    def sc_kernel(feat_hbm, ptsT_hbm, boxp_hbm, wb_hbm,
                  ofeat_hbm, oxyz_hbm, flag_hbm,
                  xyz_v, boxp_v, wb_v, filt_v, idx_v, exp_v, feat_v, xyzg_v,
                  flags_v, gsem0, gsem1, gsem2, gsem3,
                  osem0, osem1, osem2, osem3):
        wid = lax.axis_index("s") * 2 + lax.axis_index("c")
        b = wid // W_PER_B
        box0 = wid * BOXES_PER_W

        pltpu.sync_copy(ptsT_hbm.at[b], xyz_v)
        pltpu.sync_copy(boxp_hbm.at[pl.ds(box0, BOXES_PER_W)], boxp_v)
        pltpu.sync_copy(wb_hbm.at[pl.ds(wid, 1)], wb_v)

        lane = lax.iota(jnp.int32, _LANES)

        wrow = wb_v[0, :]
        xlo, xhi = wrow[0], wrow[1]
        ylo, yhi = wrow[2], wrow[3]
        zlo, zhi = wrow[4], wrow[5]

        def bb_test(base):
            x = xyz_v[0, pl.ds(base, _LANES)]
            y = xyz_v[1, pl.ds(base, _LANES)]
            z = xyz_v[2, pl.ds(base, _LANES)]
            return ((x >= xlo) & (x <= xhi) & (y >= ylo) & (y <= yhi)
                    & (z >= zlo) & (z <= zhi))

        def filt_body(ii, off):
            base = ii * (4 * _LANES)
            ms = [bb_test(base + t * _LANES) for t in range(4)]
            cs = [plsc.cumsum(m.astype(jnp.int32)) for m in ms]
            pc = [plsc.all_reduce_population_count(m) for m in ms]
            run = off
            for t in range(4):
                plsc.store_scatter(filt_v, [run + cs[t] - 1],
                                   base + t * _LANES + lane, mask=ms[t])
                run = run + pc[t]
            return run

        nf = lax.fori_loop(0, N // (4 * _LANES), filt_body,
                           jnp.zeros((_LANES,), jnp.int32))
        for t in range(4):
            plsc.store_scatter(filt_v, [nf + t * _LANES + lane],
                               jnp.zeros((_LANES,), jnp.int32))
        nf_s = nf[0]
        ntrip = (nf_s + 4 * _LANES - 1) // (4 * _LANES)

        gsems = [gsem0, gsem1, gsem2, gsem3]
        osems = [osem0, osem1, osem2, osem3]
        NBUF = 4

        def drain_feat(k):
            pltpu.make_async_copy(
                feat_v.at[k], ofeat_hbm.at[0, pl.ds(0, PIECE), :],
                osems[k]).wait()

        def make_box_body(first):
          def box_body(j, flags):
            prow = boxp_v[j, :]
            cx = prow[0]
            cy = prow[1]
            cz = prow[2]
            dx2 = prow[3]
            dy2 = prow[4]
            dz2 = prow[5]
            ca = prow[6]
            sa = prow[7]

            def test(base):
                fc = filt_v[pl.ds(base, _LANES)]
                x = plsc.load_gather(
                    xyz_v, [jnp.zeros((_LANES,), jnp.int32), fc])
                y = plsc.load_gather(
                    xyz_v, [jnp.ones((_LANES,), jnp.int32), fc])
                z = plsc.load_gather(
                    xyz_v, [jnp.full((_LANES,), 2, jnp.int32), fc])
                sx = x - cx
                sy = y - cy
                lx = sx * ca + sy * (-sa)
                ly = sx * sa + sy * ca
                m = (
                    (jnp.abs(z - cz) <= dz2)
                    & (lx > -dx2) & (lx < dx2)
                    & (ly > -dy2) & (ly < dy2)
                    & (base + lane < nf)
                )
                return fc, m

            def scan_body(ii, off):
                base = ii * (4 * _LANES)
                fm = [test(base + t * _LANES) for t in range(4)]
                cs = [plsc.cumsum(m.astype(jnp.int32)) for _, m in fm]
                pc = [plsc.all_reduce_population_count(m) for _, m in fm]
                run = off
                for t in range(4):
                    pos = run + cs[t] - 1
                    plsc.store_scatter(idx_v, [pos], fm[t][0],
                                       mask=fm[t][1] & (pos < S))
                    run = run + pc[t]
                return run

            cnt = lax.fori_loop(0, ntrip, scan_body,
                                jnp.zeros((_LANES,), jnp.int32))
            cnt_safe = jnp.maximum(cnt, 1)
            nonempty = cnt[0] > 0

            bm = box0 + j

            @pl.when(nonempty)
            def _pooled():
                gcp = [None] * NP

                def expand(q):
                    for r in range(0, PIECE, _LANES):
                        ar = q * PIECE + r + lane
                        p = jnp.where(ar < cnt, ar, lax.rem(ar, cnt_safe))
                        g = plsc.load_gather(idx_v, [p])
                        exp_v[q, pl.ds(r, _LANES)] = b * N + g
                        for d in range(3):
                            xv = plsc.load_gather(
                                xyz_v, [jnp.full((_LANES,), d, jnp.int32), g])
                            plsc.store_scatter(xyzg_v, [ar * 3 + d], xv)

                def fire(q):
                    gcp[q] = pltpu.async_copy(
                        feat_hbm.at[exp_v.at[q]], feat_v.at[q % NBUF],
                        gsems[q % NBUF])

                def out(q):
                    pltpu.async_copy(
                        feat_v.at[q % NBUF],
                        ofeat_hbm.at[bm, pl.ds(q * PIECE, PIECE), :],
                        osems[q % NBUF])

                for q in range(NBUF):
                    if not first:
                        drain_feat(q)
                    expand(q)
                    fire(q)
                for q in range(NBUF, NP):
                    expand(q)
                pltpu.sync_copy(xyzg_v, oxyz_hbm.at[bm])
                for q in range(NP):
                    gcp[q].wait()
                    out(q)
                    nq = q + NBUF
                    if nq < NP:
                        drain_feat(nq % NBUF)
                        fire(nq)

            @pl.when(jnp.logical_not(nonempty))
            def _zeros():
                zv = jnp.zeros((_LANES,), jnp.float32)

                def zx(r, _):
                    xyzg_v[pl.ds(r * _LANES, _LANES)] = zv
                    return 0

                lax.fori_loop(0, (S * 3) // _LANES, zx, 0)
                pltpu.sync_copy(xyzg_v, oxyz_hbm.at[bm])

                def zf(k):
                    def body(r, _):
                        for c0 in range(0, C, _LANES):
                            feat_v[k, r, pl.ds(c0, _LANES)] = zv
                        return 0
                    lax.fori_loop(0, PIECE, body, 0)

                for k in range(NBUF):
                    if not first:
                        drain_feat(k)
                    zf(k)
                for q in range(NP):
                    pltpu.async_copy(
                        feat_v.at[q % NBUF],
                        ofeat_hbm.at[bm, pl.ds(q * PIECE, PIECE), :],
                        osems[q % NBUF])
                for k in range(NBUF):
                    drain_feat(k)

            return jnp.where(lane == j, 1 - (cnt > 0).astype(jnp.int32),
                             flags)
          return box_body

        flags = make_box_body(True)(0, jnp.zeros((_LANES,), jnp.int32))
        flags = lax.fori_loop(1, BOXES_PER_W, make_box_body(False), flags)
        for k in range(NBUF):
            drain_feat(k)
        flags_v[...] = flags
        pltpu.sync_copy(flags_v, flag_hbm.at[pl.ds(box0, BOXES_PER_W)])

    return sc_kernel


def kernel(points, point_features, boxes3d):
    B, N, _ = points.shape
    C = point_features.shape[2]
    M = boxes3d.shape[1]
    S = _NUM_SAMPLED

    feat = point_features.reshape(B * N, C)
    ptsT = jnp.transpose(points, (0, 2, 1))

    cx = boxes3d[..., 0]
    cy = boxes3d[..., 1]
    dz = boxes3d[..., 5]
    cz = boxes3d[..., 2] + dz / 2.0
    dx2 = boxes3d[..., 3] / 2.0
    dy2 = boxes3d[..., 4] / 2.0
    dz2 = dz / 2.0
    ca = jnp.cos(-boxes3d[..., 6])
    sa = jnp.sin(-boxes3d[..., 6])
    zz = jnp.zeros_like(cx)
    boxp = jnp.stack([cx, cy, cz, dx2, dy2, dz2, ca, sa,
                      zz, zz, zz, zz, zz, zz, zz, zz],
                     axis=-1).reshape(B * M, _LANES)

    NW = 32
    r = (dx2 + dy2).reshape(NW, -1) * 1.001 + 1e-5
    cxw = cx.reshape(NW, -1)
    cyw = cy.reshape(NW, -1)
    czw = cz.reshape(NW, -1)
    dzw = dz2.reshape(NW, -1) * 1.001 + 1e-5
    xlo = jnp.min(cxw - r, axis=1)
    xhi = jnp.max(cxw + r, axis=1)
    ylo = jnp.min(cyw - r, axis=1)
    yhi = jnp.max(cyw + r, axis=1)
    zlo = jnp.min(czw - dzw, axis=1)
    zhi = jnp.max(czw + dzw, axis=1)
    zw = jnp.zeros_like(xlo)
    wb = jnp.stack([xlo, xhi, ylo, yhi, zlo, zhi,
                    zw, zw, zw, zw, zw, zw, zw, zw, zw, zw],
                   axis=-1)

    sc = _make_sc_kernel(B, N, C, M, S)
    ofeat, oxyz, flag = sc(feat, ptsT, boxp, wb)
    pooled = jnp.concatenate(
        [oxyz.reshape(B, M, S, 3), ofeat.reshape(B, M, S, C)], axis=-1)
    return pooled, flag.reshape(B, M)
